# baseline (speedup 1.0000x reference)
import functools

import jax
import jax.numpy as jnp
from jax import lax
from jax.experimental import pallas as pl
from jax.experimental.pallas import tpu as pltpu
from jax.experimental.pallas import tpu_sc as plsc

_B, _H, _W = 4, 256, 512
_D = 192
_PH, _PW = 64, 128
_CHUNK = 64
_NACC = 128
_NSUB = 32
_ROWS_PER_SUB = (_B * _PH) // _NSUB

_MAX_DISP = 192.0
_EPS = 1e-6


def _robust(diff):
    x = diff * 2.0
    x = x * x * (1.0 / 1.2) + 1.0
    x = jnp.exp(0.4 * jnp.log(x))
    return (x - 1.0) * 1.5


def _pixel_kernel(t_ref,
                  p0, p1, p2, p3, p4, p5,
                  c0, c1, c2, c3,
                  dxy_ref,
                  s0, s1, s2, s3, s4, s5,
                  out_ref):
    step = pl.program_id(0)
    t = t_ref[...]
    mask = (t < _MAX_DISP) & (t > 0.001)
    mf = mask.astype(jnp.float32)

    accs = []
    accs.append(jnp.sum(mf))

    preds = (p0, p1, p2, p3, p4, p5)
    diffs = [jnp.abs(p[...] - t) for p in preds]
    rl = 0.0
    for d in diffs:
        rl = rl + jnp.sum(_robust(d) * mf)
    accs.append(rl)

    s_num, s_den = [], []
    for i, s in enumerate((s0, s1, s2, s3, s4, s5)):
        m = mf * (diffs[i] < 1.0).astype(jnp.float32)
        tot = 0.0
        for b in range(_B):
            cross = 0.0
            for b2 in range(_B):
                cross = cross + (jnp.abs(dxy_ref[b, 0] - s[b2, 0])
                                 + jnp.abs(dxy_ref[b, 1] - s[b2, 1]))
            tot = tot + jnp.sum(m[b] * cross)
        s_num.append(tot)
        s_den.append(jnp.sum(m))
    accs.extend(s_num)
    accs.extend(s_den)

    confs = (c0, c1, c2, c3)
    conf_diff_idx = (1, 2, 4, 5)
    c_num, c_den = [], []
    for cr, di in zip(confs, conf_diff_idx):
        d = diffs[di]
        closer = (d < 1.0).astype(jnp.float32)
        further = (d > 1.5).astype(jnp.float32)
        sel = closer + further
        m = mf * sel
        cv = cr[...]
        loss = jnp.maximum(1.0 - cv, 0.0) * closer + jnp.maximum(cv, 0.0) * further
        c_num.append(jnp.sum(loss * m))
        c_den.append(jnp.sum(m))
    accs.extend(c_num)
    accs.extend(c_den)

    lane = jax.lax.broadcasted_iota(jnp.int32, (1, _NACC), 1)
    row = jnp.zeros((1, _NACC), jnp.float32)
    for i, v in enumerate(accs):
        row = jnp.where(lane == i, v, row)

    @pl.when(step == 0)
    def _():
        out_ref[...] = row

    @pl.when(step != 0)
    def _():
        out_ref[...] = out_ref[...] + row


def _pool_kernel(t_ref, out_ref):
    wi = jax.lax.broadcasted_iota(jnp.int32, (_W, _PW), 0)
    ci = jax.lax.broadcasted_iota(jnp.int32, (_W, _PW), 1)
    sks = [(wi == 4 * ci + k).astype(jnp.float32) for k in range(4)]
    for b in range(_B):
        t = t_ref[b]
        tr = jnp.max(t.reshape(_PH, 4, _W), axis=1)
        pooled = None
        for sk in sks:
            pk = jax.lax.dot(tr, sk, preferred_element_type=jnp.float32)
            pooled = pk if pooled is None else jnp.maximum(pooled, pk)
        out_ref[pl.ds(b * _PH, _PH), :] = pooled


_sc_mesh = plsc.VectorSubcoreMesh(core_axis_name="c", subcore_axis_name="s")


@functools.partial(
    pl.kernel,
    out_type=jax.ShapeDtypeStruct((3, _NSUB, 16), jnp.float32),
    mesh=_sc_mesh,
    scratch_types=[
        pltpu.VMEM((_D, _PW), jnp.float32),
        pltpu.VMEM((_PW,), jnp.float32),
        pltpu.VMEM((3, 16), jnp.float32),
    ],
)
def _sc_volume(vol_hbm, pooled_hbm, out_hbm, vol_v, prow_v, acc_v):
    wid = lax.axis_index("s") * 2 + lax.axis_index("c")
    inf16 = jnp.full((16,), jnp.inf, jnp.float32)
    zero16 = jnp.zeros((16,), jnp.float32)
    acc_sm = zero16
    acc_gt = zero16
    acc_nm = zero16

    for rr in range(_ROWS_PER_SUB):
        row = wid * _ROWS_PER_SUB + rr
        b = row // _PH
        r = row % _PH
        pltpu.sync_copy(pooled_hbm.at[row, :], prow_v)
        pltpu.sync_copy(vol_hbm.at[b, :, r, :], vol_v)

        groups = []
        for g in range(8):
            t16 = prow_v[pl.ds(g * 16, 16)]
            mask = (t16 < _MAX_DISP) & (t16 > 0.001)
            mf = jnp.where(mask, 1.0, 0.0)
            e16 = 1.0 + jnp.maximum(0.0, t16 - (_MAX_DISP - 1.0))
            groups.append((t16, mf, e16))

        def body(d, carry):
            d_f = jnp.full((16,), d, jnp.float32)
            out = []
            for g in range(8):
                t16, _, e16 = groups[g]
                mv, ph = carry[2 * g], carry[2 * g + 1]
                v = vol_v[d, pl.ds(g * 16, 16)]
                absd = jnp.abs(d_f - t16)
                keep = absd > 1.5
                mv = jnp.minimum(mv, jnp.where(keep, v, inf16))
                ph = ph + v * jnp.maximum(0.0, e16 - absd)
                out.extend((mv, ph))
            return tuple(out)

        carry = lax.fori_loop(0, _D, body, (inf16, zero16) * 8)

        for g in range(8):
            t16, mf, _ = groups[g]
            mv, ph = carry[2 * g], carry[2 * g + 1]
            acc_sm = acc_sm + mf
            acc_gt = acc_gt + ph * mf
            acc_nm = acc_nm + jnp.maximum(1.0 - mv, 0.0) * mf

    acc_v[0, :] = acc_sm
    acc_v[1, :] = acc_gt
    acc_v[2, :] = acc_nm
    pltpu.sync_copy(acc_v, out_hbm.at[:, wid, :])


def _combine_kernel(acc_ref, sc_ref, out_ref):
    sm = jnp.sum(sc_ref[0])
    gt = jnp.sum(sc_ref[1])
    nm = jnp.sum(sc_ref[2])
    scale_l = acc_ref[0, 1] / (acc_ref[0, 0] + _EPS)
    slant_l = 0.0
    for i in range(6):
        slant_l = slant_l + acc_ref[0, 2 + i] / (acc_ref[0, 8 + i] + _EPS)
    conf_l = 0.0
    for i in range(4):
        conf_l = conf_l + acc_ref[0, 14 + i] / (acc_ref[0, 18 + i] + _EPS)
    init_l = (gt + nm) / (sm + _EPS)
    out_ref[0, 0] = scale_l + init_l + slant_l + conf_l


def kernel(preds_0, preds_1, preds_2, preds_coarse_0, preds_coarse_1,
           preds_coarse_2, slant_0, slant_1, slant_2, slant_coarse_0,
           slant_coarse_1, slant_coarse_2, conf_0, conf_1, conf_coarse_0,
           conf_coarse_1, volume_0, target, dxygt):
    pooled = pl.pallas_call(
        _pool_kernel,
        out_shape=jax.ShapeDtypeStruct((_B * _PH, _PW), jnp.float32),
    )(target)

    sc_part = _sc_volume(volume_0, pooled)

    pix_inputs = [target,
                  preds_0, preds_1, preds_2,
                  preds_coarse_0, preds_coarse_1, preds_coarse_2,
                  conf_0, conf_1, conf_coarse_0, conf_coarse_1,
                  dxygt,
                  slant_0, slant_1, slant_2,
                  slant_coarse_0, slant_coarse_1, slant_coarse_2]

    n_steps = _H // _CHUNK
    in_spec3 = pl.BlockSpec((_B, _CHUNK, _W), lambda i: (0, i, 0))
    in_spec4 = pl.BlockSpec((_B, 2, _CHUNK, _W), lambda i: (0, 0, i, 0))
    specs = [in_spec3] * 11 + [in_spec4] * 7
    acc = pl.pallas_call(
        _pixel_kernel,
        grid=(n_steps,),
        in_specs=specs,
        out_specs=pl.BlockSpec((1, _NACC), lambda i: (0, 0)),
        out_shape=jax.ShapeDtypeStruct((1, _NACC), jnp.float32),
    )(*pix_inputs)

    out = pl.pallas_call(
        _combine_kernel,
        in_specs=[pl.BlockSpec(memory_space=pltpu.SMEM),
                  pl.BlockSpec(memory_space=pltpu.VMEM)],
        out_specs=pl.BlockSpec(memory_space=pltpu.SMEM),
        out_shape=jax.ShapeDtypeStruct((1, 1), jnp.float32),
    )(acc, sc_part)

    return out[0, 0]
